# Initial kernel scaffold; baseline (speedup 1.0000x reference)
import jax
import jax.numpy as jnp
from jax.experimental import pallas as pl


def kernel(x, table):
    raise NotImplementedError("write your pallas kernel here")


# speedup vs baseline: 1.6838x; 1.6838x over previous
import functools

import jax
import jax.numpy as jnp
from jax import lax
from jax.experimental import pallas as pl
from jax.experimental.pallas import tpu as pltpu
from jax.experimental.pallas import tpu_sc as plsc

NUM_CORES = 2
NUM_SUBCORES = 16
NW = NUM_CORES * NUM_SUBCORES
CHUNK = 128
EMB_DIM = 64


def _make_lookup(n_chunks: int):
    mesh = plsc.VectorSubcoreMesh(core_axis_name="c", subcore_axis_name="s")

    @functools.partial(
        pl.kernel,
        mesh=mesh,
        out_type=jax.ShapeDtypeStruct((NW, n_chunks, CHUNK, EMB_DIM), jnp.float32),
        scratch_types=[
            pltpu.VMEM((n_chunks, CHUNK), jnp.int32),
            pltpu.VMEM((CHUNK, EMB_DIM), jnp.float32),
            pltpu.SemaphoreType.DMA,
        ],
        compiler_params=pltpu.CompilerParams(use_tc_tiling_on_sc=False),
    )
    def lookup(idx_hbm, table_hbm, out_hbm, idx_v, buf, sem):
        wid = lax.axis_index("s") * NUM_CORES + lax.axis_index("c")
        pltpu.sync_copy(idx_hbm.at[wid], idx_v)

        def body(j, carry):
            pltpu.async_copy(table_hbm.at[idx_v.at[j]], buf, sem).wait()
            pltpu.sync_copy(buf, out_hbm.at[wid, j])
            return carry

        lax.fori_loop(0, n_chunks, body, 0)

    return lookup


def kernel(x, table):
    b, s = x.shape[1], x.shape[2]
    total = b * s
    n_chunks = total // (NW * CHUNK)
    idx = jnp.reshape(x[0].astype(jnp.int32), (NW, n_chunks, CHUNK))
    out = _make_lookup(n_chunks)(idx, table)
    return jnp.reshape(out, (b, s, EMB_DIM))


# speedup vs baseline: 1.8723x; 1.1120x over previous
import functools

import jax
import jax.numpy as jnp
from jax import lax
from jax.experimental import pallas as pl
from jax.experimental.pallas import tpu as pltpu
from jax.experimental.pallas import tpu_sc as plsc

NUM_CORES = 2
NUM_SUBCORES = 16
NW = NUM_CORES * NUM_SUBCORES
CHUNK = 128
EMB_DIM = 64
NBUF = 8
K = 4


def _make_lookup(n_chunks: int):
    assert n_chunks % NBUF == 0 and n_chunks > NBUF
    mesh = plsc.VectorSubcoreMesh(core_axis_name="c", subcore_axis_name="s")

    @functools.partial(
        pl.kernel,
        mesh=mesh,
        out_type=jax.ShapeDtypeStruct((NW, n_chunks, CHUNK, EMB_DIM), jnp.float32),
        scratch_types=[
            pltpu.VMEM((n_chunks, CHUNK), jnp.int32),
            pltpu.VMEM((NBUF, CHUNK, EMB_DIM), jnp.float32),
            pltpu.SemaphoreType.DMA((NBUF,)),
            pltpu.SemaphoreType.DMA((NBUF,)),
        ],
        compiler_params=pltpu.CompilerParams(use_tc_tiling_on_sc=False),
    )
    def lookup(idx_hbm, table_hbm, out_hbm, idx_v, buf, gsem, wsem):
        wid = lax.axis_index("s") * NUM_CORES + lax.axis_index("c")
        pltpu.sync_copy(idx_hbm.at[wid], idx_v)

        for b in range(K):
            pltpu.async_copy(table_hbm.at[idx_v.at[b]], buf.at[b], gsem.at[b])

        def body(j, carry):
            nj = j + K
            ns = lax.rem(nj, NBUF)

            @pl.when(jnp.logical_and(nj < n_chunks, nj >= NBUF))
            def _():
                pltpu.make_async_copy(
                    buf.at[ns], out_hbm.at[wid, nj - NBUF], wsem.at[ns]
                ).wait()

            @pl.when(nj < n_chunks)
            def _():
                pltpu.async_copy(table_hbm.at[idx_v.at[nj]], buf.at[ns], gsem.at[ns])

            b = lax.rem(j, NBUF)
            pltpu.make_async_copy(
                table_hbm.at[idx_v.at[j]], buf.at[b], gsem.at[b]
            ).wait()
            pltpu.async_copy(buf.at[b], out_hbm.at[wid, j], wsem.at[b])
            return carry

        lax.fori_loop(0, n_chunks, body, 0)

        for b in range(NBUF):
            pltpu.make_async_copy(
                buf.at[b], out_hbm.at[wid, n_chunks - NBUF + b], wsem.at[b]
            ).wait()

    return lookup


def kernel(x, table):
    b, s = x.shape[1], x.shape[2]
    total = b * s
    n_chunks = total // (NW * CHUNK)
    idx = jnp.reshape(x[0].astype(jnp.int32), (NW, n_chunks, CHUNK))
    out = _make_lookup(n_chunks)(idx, table)
    return jnp.reshape(out, (b, s, EMB_DIM))


# speedup vs baseline: 2.1758x; 1.1621x over previous
import functools

import jax
import jax.numpy as jnp
from jax import lax
from jax.experimental import pallas as pl
from jax.experimental.layout import Layout, with_layout_constraint
from jax.experimental.pallas import tpu as pltpu
from jax.experimental.pallas import tpu_sc as plsc

NUM_CORES = 2
NUM_SUBCORES = 16
NW = NUM_CORES * NUM_SUBCORES
CHUNK = 128
EMB_DIM = 64
NBUF = 8
K = 4


def _make_lookup(n_chunks: int):
    assert n_chunks % NBUF == 0 and n_chunks > NBUF
    mesh = plsc.VectorSubcoreMesh(core_axis_name="c", subcore_axis_name="s")

    @functools.partial(
        pl.kernel,
        mesh=mesh,
        out_type=jax.ShapeDtypeStruct((NW, n_chunks, CHUNK, EMB_DIM), jnp.float32),
        scratch_types=[
            pltpu.VMEM((n_chunks, CHUNK), jnp.int32),
            pltpu.VMEM((NBUF, CHUNK, EMB_DIM), jnp.float32),
            pltpu.SemaphoreType.DMA((NBUF,)),
            pltpu.SemaphoreType.DMA((NBUF,)),
        ],
        compiler_params=pltpu.CompilerParams(use_tc_tiling_on_sc=False),
    )
    def lookup(idx_hbm, table_hbm, out_hbm, idx_v, buf, gsem, wsem):
        wid = lax.axis_index("s") * NUM_CORES + lax.axis_index("c")
        pltpu.sync_copy(idx_hbm.at[wid], idx_v)

        for b in range(K):
            pltpu.async_copy(table_hbm.at[idx_v.at[b]], buf.at[b], gsem.at[b])

        def body(j, carry):
            nj = j + K
            ns = lax.rem(nj, NBUF)

            @pl.when(jnp.logical_and(nj < n_chunks, nj >= NBUF))
            def _():
                pltpu.make_async_copy(
                    buf.at[ns], out_hbm.at[wid, nj - NBUF], wsem.at[ns]
                ).wait()

            @pl.when(nj < n_chunks)
            def _():
                pltpu.async_copy(table_hbm.at[idx_v.at[nj]], buf.at[ns], gsem.at[ns])

            b = lax.rem(j, NBUF)
            pltpu.make_async_copy(
                table_hbm.at[idx_v.at[j]], buf.at[b], gsem.at[b]
            ).wait()
            pltpu.async_copy(buf.at[b], out_hbm.at[wid, j], wsem.at[b])
            return carry

        lax.fori_loop(0, n_chunks, body, 0)

        for b in range(NBUF):
            pltpu.make_async_copy(
                buf.at[b], out_hbm.at[wid, n_chunks - NBUF + b], wsem.at[b]
            ).wait()

    return lookup


def kernel(x, table):
    b, s = x.shape[1], x.shape[2]
    total = b * s
    n_chunks = total // (NW * CHUNK)
    idx = jnp.reshape(x[0].astype(jnp.int32), (NW, n_chunks, CHUNK))
    out = _make_lookup(n_chunks)(idx, table)
    y = jnp.reshape(out, (b, s, EMB_DIM))
    return with_layout_constraint(
        y, Layout(major_to_minor=(0, 1, 2), tiling=((EMB_DIM,),))
    )
